# Initial kernel scaffold; baseline (speedup 1.0000x reference)
import jax
import jax.numpy as jnp
from jax.experimental import pallas as pl


def kernel(outputs, targets):
    raise NotImplementedError("write your pallas kernel here")


# speedup vs baseline: 36.8194x; 36.8194x over previous
import functools

import jax
import jax.numpy as jnp
from jax import lax
from jax.experimental import pallas as pl
from jax.experimental.pallas import tpu as pltpu
from jax.experimental.pallas import tpu_sc as plsc

D = 16
IGNORE_VALUE = -1000.0
BIG = float(jnp.finfo(jnp.float32).max)

NUM_CORES = 2
NUM_SUBCORES = 16
NUM_WORKERS = NUM_CORES * NUM_SUBCORES
CHUNK_ROWS = 2048


def _sc_body(rows_per_worker, n_chunks, out_hbm, tgt_hbm, loss_hbm,
             o_v, t_v, acc_v):
    cid = lax.axis_index("c")
    sid = lax.axis_index("s")
    wid = sid * NUM_CORES + cid
    row0 = wid * rows_per_worker
    iota = lax.iota(jnp.int32, 16)
    tiles_per_chunk = CHUNK_ROWS // 16

    def chunk_body(c, acc):
        ebase = (row0 + c * CHUNK_ROWS) * 16
        pltpu.sync_copy(out_hbm.at[pl.ds(ebase, CHUNK_ROWS * 16)], o_v)
        pltpu.sync_copy(tgt_hbm.at[pl.ds(ebase, CHUNK_ROWS * 16)], t_v)

        def tile_body(t, acc):
            base = t * 256 + iota * 16
            o = [plsc.load_gather(o_v, [base + j]) for j in range(D)]
            for i in range(D):
                sel = plsc.load_gather(t_v, [base + i])
                curmin = jnp.abs(o[0] - sel)
                curidx = jnp.zeros((16,), jnp.int32)
                for j in range(1, D):
                    d = jnp.abs(o[j] - sel)
                    m = d < curmin
                    curmin = jnp.minimum(d, curmin)
                    curidx = jnp.where(m, j, curidx)
                for j in range(D):
                    o[j] = jnp.where(curidx == j, BIG, o[j])
                acc = acc + jnp.where(sel != IGNORE_VALUE,
                                      curmin * curmin, 0.0)
            return acc

        return lax.fori_loop(0, tiles_per_chunk, tile_body, acc)

    acc = lax.fori_loop(0, n_chunks, chunk_body,
                        jnp.zeros((16,), jnp.float32))
    acc_v[...] = acc
    pltpu.sync_copy(acc_v, loss_hbm.at[wid])


def kernel(outputs, targets):
    B, Dn = outputs.shape
    assert Dn == D and B % (NUM_WORKERS * CHUNK_ROWS) == 0
    rows_per_worker = B // NUM_WORKERS
    n_chunks = rows_per_worker // CHUNK_ROWS

    mesh = plsc.VectorSubcoreMesh(
        core_axis_name="c", subcore_axis_name="s",
        num_cores=NUM_CORES, num_subcores=NUM_SUBCORES)
    partials = pl.kernel(
        functools.partial(_sc_body, rows_per_worker, n_chunks),
        out_type=jax.ShapeDtypeStruct((NUM_WORKERS, 16), jnp.float32),
        mesh=mesh,
        compiler_params=pltpu.CompilerParams(needs_layout_passes=False),
        scratch_types=[
            pltpu.VMEM((CHUNK_ROWS * 16,), jnp.float32),
            pltpu.VMEM((CHUNK_ROWS * 16,), jnp.float32),
            pltpu.VMEM((16,), jnp.float32),
        ],
    )(outputs.reshape(-1), targets.reshape(-1))
    return jnp.sum(partials) / (B * D)


# speedup vs baseline: 38.3391x; 1.0413x over previous
import functools

import numpy as np

import jax
import jax.numpy as jnp
from jax import lax
from jax.experimental import pallas as pl
from jax.experimental.pallas import tpu as pltpu
from jax.experimental.pallas import tpu_sc as plsc

D = 16
IGNORE_VALUE = -1000.0
BIG = float(np.finfo(np.float32).max)
ABS_TRUNC_MASK = np.uint32(0x7FFFFFF0)

NUM_CORES = 2
NUM_SUBCORES = 16
NUM_WORKERS = NUM_CORES * NUM_SUBCORES
CHUNK_ROWS = 2048


def _sc_body(rows_per_worker, n_chunks, out_hbm, tgt_hbm, loss_hbm,
             o_v, t_v, acc_v):
    cid = lax.axis_index("c")
    sid = lax.axis_index("s")
    wid = sid * NUM_CORES + cid
    row0 = wid * rows_per_worker
    iota = lax.iota(jnp.int32, 16)
    tiles_per_chunk = CHUNK_ROWS // 16

    def chunk_body(c, acc):
        ebase = (row0 + c * CHUNK_ROWS) * 16
        pltpu.sync_copy(out_hbm.at[pl.ds(ebase, CHUNK_ROWS * 16)], o_v)
        pltpu.sync_copy(tgt_hbm.at[pl.ds(ebase, CHUNK_ROWS * 16)], t_v)

        def tile_body(t, acc):
            base = t * 256 + iota * 16
            o = [plsc.load_gather(o_v, [base + j]) for j in range(D)]
            for i in range(D):
                sel = plsc.load_gather(t_v, [base + i])
                keys = [
                    (lax.bitcast_convert_type(o[j] - sel, jnp.uint32)
                     & ABS_TRUNC_MASK) | np.uint32(j)
                    for j in range(D)
                ]
                k = keys
                while len(k) > 1:
                    k = [jnp.minimum(k[2 * a], k[2 * a + 1])
                         for a in range(len(k) // 2)]
                curkey = k[0]
                for j in range(D):
                    o[j] = jnp.where(keys[j] == curkey, BIG, o[j])
                dmin = lax.bitcast_convert_type(
                    curkey & ABS_TRUNC_MASK, jnp.float32)
                acc = acc + jnp.where(sel != IGNORE_VALUE,
                                      dmin * dmin, 0.0)
            return acc

        return lax.fori_loop(0, tiles_per_chunk, tile_body, acc)

    acc = lax.fori_loop(0, n_chunks, chunk_body,
                        jnp.zeros((16,), jnp.float32))
    acc_v[...] = acc
    pltpu.sync_copy(acc_v, loss_hbm.at[wid])


def kernel(outputs, targets):
    B, Dn = outputs.shape
    assert Dn == D and B % (NUM_WORKERS * CHUNK_ROWS) == 0
    rows_per_worker = B // NUM_WORKERS
    n_chunks = rows_per_worker // CHUNK_ROWS

    mesh = plsc.VectorSubcoreMesh(
        core_axis_name="c", subcore_axis_name="s",
        num_cores=NUM_CORES, num_subcores=NUM_SUBCORES)
    partials = pl.kernel(
        functools.partial(_sc_body, rows_per_worker, n_chunks),
        out_type=jax.ShapeDtypeStruct((NUM_WORKERS, 16), jnp.float32),
        mesh=mesh,
        compiler_params=pltpu.CompilerParams(needs_layout_passes=False),
        scratch_types=[
            pltpu.VMEM((CHUNK_ROWS * 16,), jnp.float32),
            pltpu.VMEM((CHUNK_ROWS * 16,), jnp.float32),
            pltpu.VMEM((16,), jnp.float32),
        ],
    )(outputs.reshape(-1), targets.reshape(-1))
    return jnp.sum(partials) / (B * D)
